# baseline (speedup 1.0000x reference)
import functools

import jax
import jax.numpy as jnp
from jax import lax
from jax.experimental import pallas as pl
from jax.experimental.pallas import tpu as pltpu
from jax.experimental.pallas import tpu_sc as plsc

_LANES = 16
_CH = 64
_NC = 2
_NS = 16
_SEGW = 32


def _tc_mlp_body(w1x_ref, w1h_ref, w2_ref, b1_ref, b2_ref, x_ref, h_ref,
                 out_ref):
    xb = x_ref[...].astype(jnp.bfloat16)
    hb = h_ref[...].astype(jnp.bfloat16)
    hidden = jnp.maximum(
        jnp.dot(xb, w1x_ref[...], preferred_element_type=jnp.float32)
        + jnp.dot(hb, w1h_ref[...], preferred_element_type=jnp.float32)
        + b1_ref[...], 0.0)
    out_ref[...] = jnp.dot(hidden.astype(jnp.bfloat16), w2_ref[...],
                           preferred_element_type=jnp.float32) + b2_ref[...]


def _extract(vec_ref, j):
    v = vec_ref[pl.ds(j, _LANES)]
    return v[0]


def _sc_segsum_body(h_hbm, ptr_hbm, out_hbm, ptr_v, buf_v, outbuf_v):
    wid = lax.axis_index("s") * _NC + lax.axis_index("c")
    s0 = wid * _SEGW
    pltpu.sync_copy(ptr_hbm.at[pl.ds(s0, 4 * _LANES)], ptr_v)

    def seg_body(j, carry):
        start = _extract(ptr_v, j)
        end = _extract(ptr_v, j + 1)
        abase = (start // 8) * 8
        nch = (end - abase + _CH - 1) // _CH

        def chunk_body(c, accs):
            off = abase + c * _CH
            pltpu.sync_copy(h_hbm.at[pl.ds(off, _CH)], buf_v)
            rlo = jnp.maximum(start - off, 0)
            rhi = jnp.minimum(end - off, _CH)

            def row_body(r, accs):
                return tuple(accs[k] + buf_v[r, pl.ds(k * _LANES, _LANES)]
                             for k in range(8))

            return lax.fori_loop(rlo, rhi, row_body, accs)

        zeros = tuple(jnp.zeros((_LANES,), jnp.float32) for _ in range(8))
        accs = lax.fori_loop(0, nch, chunk_body, zeros)
        for k in range(8):
            outbuf_v[j, pl.ds(k * _LANES, _LANES)] = accs[k]
        return carry

    lax.fori_loop(0, _SEGW, seg_body, 0)
    pltpu.sync_copy(outbuf_v, out_hbm.at[pl.ds(s0, _SEGW)])


def kernel(h_node, x, ptr, W1, b1, W2, b2):
    n, embed_dim = h_node.shape
    nfeat = x.shape[1]
    nseg = ptr.shape[0] - 1
    hidden_dim = W1.shape[1]

    block_rows = 512
    grid = (n // block_rows,)
    w1x = W1[:nfeat].astype(jnp.bfloat16)
    w1h = W1[nfeat:].astype(jnp.bfloat16)
    w2 = W2.astype(jnp.bfloat16)

    h = pl.pallas_call(
        _tc_mlp_body,
        grid=grid,
        in_specs=[
            pl.BlockSpec((nfeat, hidden_dim), lambda i: (0, 0)),
            pl.BlockSpec((embed_dim, hidden_dim), lambda i: (0, 0)),
            pl.BlockSpec((hidden_dim, embed_dim), lambda i: (0, 0)),
            pl.BlockSpec((1, hidden_dim), lambda i: (0, 0)),
            pl.BlockSpec((1, embed_dim), lambda i: (0, 0)),
            pl.BlockSpec((block_rows, nfeat), lambda i: (i, 0)),
            pl.BlockSpec((block_rows, embed_dim), lambda i: (i, 0)),
        ],
        out_specs=pl.BlockSpec((block_rows, embed_dim), lambda i: (i, 0)),
        out_shape=jax.ShapeDtypeStruct((n + _CH, embed_dim), jnp.float32),
        compiler_params=pltpu.CompilerParams(
            dimension_semantics=("arbitrary",),
        ),
    )(w1x, w1h, w2, b1.reshape(1, -1), b2.reshape(1, -1), x, h_node)

    nw = _NC * _NS
    nseg_pad = nw * _SEGW
    ptr32 = ptr.astype(jnp.int32)
    ptr_pad = jnp.concatenate(
        [ptr32, jnp.full((nseg_pad + 4 * _LANES - (nseg + 1),), n, jnp.int32)])

    mesh = plsc.VectorSubcoreMesh(core_axis_name="c", subcore_axis_name="s",
                                  num_cores=_NC, num_subcores=_NS)
    sc_out = pl.kernel(
        _sc_segsum_body,
        out_type=jax.ShapeDtypeStruct((nseg_pad, embed_dim), jnp.float32),
        mesh=mesh,
        scratch_types=[
            pltpu.VMEM((4 * _LANES,), jnp.int32),
            pltpu.VMEM((_CH, embed_dim), jnp.float32),
            pltpu.VMEM((_SEGW, embed_dim), jnp.float32),
        ],
    )(h, ptr_pad)
    return sc_out[:nseg]


# speedup vs baseline: 1.9818x; 1.9818x over previous
import functools

import jax
import jax.numpy as jnp
from jax import lax
from jax.experimental import pallas as pl
from jax.experimental.pallas import tpu as pltpu
from jax.experimental.pallas import tpu_sc as plsc

_LANES = 16
_CH = 64
_NC = 2
_NS = 16
_SEGW = 32


def _tc_mlp_body(w1x_ref, w1h_ref, w2_ref, b1_ref, b2_ref, x_ref, h_ref,
                 out_ref):
    xb = x_ref[...].astype(jnp.bfloat16)
    hb = h_ref[...].astype(jnp.bfloat16)
    hidden = jnp.maximum(
        jnp.dot(xb, w1x_ref[...], preferred_element_type=jnp.float32)
        + jnp.dot(hb, w1h_ref[...], preferred_element_type=jnp.float32)
        + b1_ref[...], 0.0)
    out_ref[...] = jnp.dot(hidden.astype(jnp.bfloat16), w2_ref[...],
                           preferred_element_type=jnp.float32) + b2_ref[...]


def _extract(vec_ref, j):
    v = vec_ref[pl.ds(j, _LANES)]
    return v[0]


def _sc_segsum_body(h_hbm, ptr_hbm, out_hbm, ptr_v, buf0_v, buf1_v, outbuf_v,
                    sem0, sem1):
    wid = lax.axis_index("s") * _NC + lax.axis_index("c")
    s0 = wid * _SEGW
    pltpu.sync_copy(ptr_hbm.at[pl.ds(s0, 4 * _LANES)], ptr_v)

    def _fetch(off, buf, sem):
        pltpu.make_async_copy(h_hbm.at[pl.ds(off, _CH)], buf, sem).start()

    def _accum(buf, off, start, end, accs):
        rlo = jnp.maximum(start - off, 0)
        rhi = jnp.minimum(end - off, _CH)

        def row_body(r, accs):
            return tuple(accs[k] + buf[r, pl.ds(k * _LANES, _LANES)]
                         for k in range(8))

        return lax.fori_loop(rlo, rhi, row_body, accs)

    def seg_body(j, carry):
        start = _extract(ptr_v, j)
        end = _extract(ptr_v, j + 1)
        abase = (start // 8) * 8
        nch = (end - abase + _CH - 1) // _CH

        @pl.when(nch > 0)
        def _():
            _fetch(abase, buf0_v, sem0)

        def pair_body(cc, accs):
            c0 = 2 * cc
            off0 = abase + c0 * _CH
            pltpu.make_async_copy(h_hbm.at[pl.ds(off0, _CH)], buf0_v,
                                  sem0).wait()
            odd = c0 + 1 < nch

            @pl.when(odd)
            def _():
                _fetch(off0 + _CH, buf1_v, sem1)

            accs = _accum(buf0_v, off0, start, end, accs)

            @pl.when(odd)
            def _():
                pltpu.make_async_copy(h_hbm.at[pl.ds(off0 + _CH, _CH)],
                                      buf1_v, sem1).wait()

            @pl.when(c0 + 2 < nch)
            def _():
                _fetch(off0 + 2 * _CH, buf0_v, sem0)

            return _accum(buf1_v, off0 + _CH, start, end, accs)

        zeros = tuple(jnp.zeros((_LANES,), jnp.float32) for _ in range(8))
        accs = lax.fori_loop(0, (nch + 1) // 2, pair_body, zeros)
        for k in range(8):
            outbuf_v[j, pl.ds(k * _LANES, _LANES)] = accs[k]
        return carry

    lax.fori_loop(0, _SEGW, seg_body, 0)
    pltpu.sync_copy(outbuf_v, out_hbm.at[pl.ds(s0, _SEGW)])


def kernel(h_node, x, ptr, W1, b1, W2, b2):
    n, embed_dim = h_node.shape
    nfeat = x.shape[1]
    nseg = ptr.shape[0] - 1
    hidden_dim = W1.shape[1]

    block_rows = 16000
    grid = (n // block_rows,)
    w1x = W1[:nfeat].astype(jnp.bfloat16)
    w1h = W1[nfeat:].astype(jnp.bfloat16)
    w2 = W2.astype(jnp.bfloat16)

    h = pl.pallas_call(
        _tc_mlp_body,
        grid=grid,
        in_specs=[
            pl.BlockSpec((nfeat, hidden_dim), lambda i: (0, 0)),
            pl.BlockSpec((embed_dim, hidden_dim), lambda i: (0, 0)),
            pl.BlockSpec((hidden_dim, embed_dim), lambda i: (0, 0)),
            pl.BlockSpec((1, hidden_dim), lambda i: (0, 0)),
            pl.BlockSpec((1, embed_dim), lambda i: (0, 0)),
            pl.BlockSpec((block_rows, nfeat), lambda i: (i, 0)),
            pl.BlockSpec((block_rows, embed_dim), lambda i: (i, 0)),
        ],
        out_specs=pl.BlockSpec((block_rows, embed_dim), lambda i: (i, 0)),
        out_shape=jax.ShapeDtypeStruct((n + _CH, embed_dim), jnp.float32),
        compiler_params=pltpu.CompilerParams(
            dimension_semantics=("parallel",),
        ),
    )(w1x, w1h, w2, b1.reshape(1, -1), b2.reshape(1, -1), x, h_node)

    nw = _NC * _NS
    nseg_pad = nw * _SEGW
    ptr32 = ptr.astype(jnp.int32)
    ptr_pad = jnp.concatenate(
        [ptr32, jnp.full((nseg_pad + 4 * _LANES - (nseg + 1),), n, jnp.int32)])

    mesh = plsc.VectorSubcoreMesh(core_axis_name="c", subcore_axis_name="s",
                                  num_cores=_NC, num_subcores=_NS)
    sc_out = pl.kernel(
        _sc_segsum_body,
        out_type=jax.ShapeDtypeStruct((nseg_pad, embed_dim), jnp.float32),
        mesh=mesh,
        scratch_types=[
            pltpu.VMEM((4 * _LANES,), jnp.int32),
            pltpu.VMEM((_CH, embed_dim), jnp.float32),
            pltpu.VMEM((_CH, embed_dim), jnp.float32),
            pltpu.VMEM((_SEGW, embed_dim), jnp.float32),
            pltpu.SemaphoreType.DMA,
            pltpu.SemaphoreType.DMA,
        ],
    )(h, ptr_pad)
    return sc_out[:nseg]


# speedup vs baseline: 2.4029x; 1.2125x over previous
import functools

import jax
import jax.numpy as jnp
from jax import lax
from jax.experimental import pallas as pl
from jax.experimental.pallas import tpu as pltpu
from jax.experimental.pallas import tpu_sc as plsc

_LANES = 16
_CH = 64
_NC = 2
_NS = 16
_NW = _NC * _NS
_NSEG_PAD = 1024
_PTR_BUF = _NSEG_PAD + 2 * _LANES
_FB = 32


def _tc_mlp_body(w1x_ref, w1h_ref, w2_ref, b1_ref, b2_ref, x_ref, h_ref,
                 out_ref):
    xb = x_ref[...].astype(jnp.bfloat16)
    hb = h_ref[...].astype(jnp.bfloat16)
    hidden = jnp.maximum(
        jnp.dot(xb, w1x_ref[...], preferred_element_type=jnp.float32)
        + jnp.dot(hb, w1h_ref[...], preferred_element_type=jnp.float32)
        + b1_ref[...], 0.0)
    out_ref[...] = jnp.dot(hidden.astype(jnp.bfloat16), w2_ref[...],
                           preferred_element_type=jnp.float32) + b2_ref[...]


def _extract(vec_ref, j):
    v = vec_ref[pl.ds(j, _LANES)]
    return v[0]


def _count_below(ptr_v, thresh):
    base = jnp.int32(0)
    step = _NSEG_PAD
    while step >= 1:
        nxt = base + step
        safe = jnp.minimum(nxt, _PTR_BUF - _LANES)
        v = _extract(ptr_v, safe)
        take = jnp.logical_and(v < thresh, nxt <= _NSEG_PAD)
        base = jnp.where(take, nxt, base)
        step //= 2
    return base


def _sc_segsum_body(h_hbm, ptr_hbm, out_hbm, ptr_v, buf0_v, buf1_v, outbuf_v,
                    idx_v, sem0, sem1):
    wid = lax.axis_index("s") * _NC + lax.axis_index("c")
    n_rows = h_hbm.shape[0] - _CH
    rw = n_rows // _NW
    pltpu.sync_copy(ptr_hbm, ptr_v)

    t_lo = wid * rw
    t_hi = (wid + 1) * rw
    f_lo = _count_below(ptr_v, t_lo)
    f_hi = jnp.where(wid == _NW - 1, _NSEG_PAD, _count_below(ptr_v, t_hi))
    nsegs = f_hi - f_lo

    iota = lax.broadcasted_iota(jnp.int32, (_LANES,), 0)
    dump = _NSEG_PAD + wid

    def _flush(batch, count):
        base = f_lo + batch * _FB
        for half in range(_FB // _LANES):
            ids = base + half * _LANES + iota
            valid = (half * _LANES + iota) < count
            idx_v[pl.ds(half * _LANES, _LANES)] = jnp.where(valid, ids, dump)
        pltpu.sync_copy(outbuf_v, out_hbm.at[idx_v])

    def _fetch(off, buf, sem):
        pltpu.make_async_copy(h_hbm.at[pl.ds(off, _CH)], buf, sem).start()

    def _accum(buf, off, start, end, accs):
        rlo = jnp.maximum(start - off, 0)
        rhi = jnp.minimum(end - off, _CH)

        def row_body(r, accs):
            return tuple(accs[k] + buf[r, pl.ds(k * _LANES, _LANES)]
                         for k in range(8))

        return lax.fori_loop(rlo, rhi, row_body, accs)

    def seg_body(j, carry):
        seg = f_lo + j
        start = _extract(ptr_v, seg)
        end = _extract(ptr_v, seg + 1)
        abase = (start // 8) * 8
        nch = (end - abase + _CH - 1) // _CH

        @pl.when(nch > 0)
        def _():
            _fetch(abase, buf0_v, sem0)

        def pair_body(cc, accs):
            c0 = 2 * cc
            off0 = abase + c0 * _CH
            pltpu.make_async_copy(h_hbm.at[pl.ds(off0, _CH)], buf0_v,
                                  sem0).wait()
            odd = c0 + 1 < nch

            @pl.when(odd)
            def _():
                _fetch(off0 + _CH, buf1_v, sem1)

            accs = _accum(buf0_v, off0, start, end, accs)

            @pl.when(odd)
            def _():
                pltpu.make_async_copy(h_hbm.at[pl.ds(off0 + _CH, _CH)],
                                      buf1_v, sem1).wait()

            @pl.when(c0 + 2 < nch)
            def _():
                _fetch(off0 + 2 * _CH, buf0_v, sem0)

            return _accum(buf1_v, off0 + _CH, start, end, accs)

        zeros = tuple(jnp.zeros((_LANES,), jnp.float32) for _ in range(8))
        accs = lax.fori_loop(0, (nch + 1) // 2, pair_body, zeros)
        slot = jnp.bitwise_and(j, _FB - 1)
        for k in range(8):
            outbuf_v[slot, pl.ds(k * _LANES, _LANES)] = accs[k]

        @pl.when(slot == _FB - 1)
        def _():
            _flush(j // _FB, _FB)

        return carry

    lax.fori_loop(0, nsegs, seg_body, 0)
    rem = jnp.bitwise_and(nsegs, _FB - 1)

    @pl.when(rem > 0)
    def _():
        _flush(nsegs // _FB, rem)


def kernel(h_node, x, ptr, W1, b1, W2, b2):
    n, embed_dim = h_node.shape
    nfeat = x.shape[1]
    nseg = ptr.shape[0] - 1
    hidden_dim = W1.shape[1]

    block_rows = 16000
    grid = (n // block_rows,)
    w1x = W1[:nfeat].astype(jnp.bfloat16)
    w1h = W1[nfeat:].astype(jnp.bfloat16)
    w2 = W2.astype(jnp.bfloat16)

    h = pl.pallas_call(
        _tc_mlp_body,
        grid=grid,
        in_specs=[
            pl.BlockSpec((nfeat, hidden_dim), lambda i: (0, 0)),
            pl.BlockSpec((embed_dim, hidden_dim), lambda i: (0, 0)),
            pl.BlockSpec((hidden_dim, embed_dim), lambda i: (0, 0)),
            pl.BlockSpec((1, hidden_dim), lambda i: (0, 0)),
            pl.BlockSpec((1, embed_dim), lambda i: (0, 0)),
            pl.BlockSpec((block_rows, nfeat), lambda i: (i, 0)),
            pl.BlockSpec((block_rows, embed_dim), lambda i: (i, 0)),
        ],
        out_specs=pl.BlockSpec((block_rows, embed_dim), lambda i: (i, 0)),
        out_shape=jax.ShapeDtypeStruct((n + _CH, embed_dim), jnp.float32),
        compiler_params=pltpu.CompilerParams(
            dimension_semantics=("parallel",),
        ),
    )(w1x, w1h, w2, b1.reshape(1, -1), b2.reshape(1, -1), x, h_node)

    ptr32 = ptr.astype(jnp.int32)
    ptr_pad = jnp.concatenate(
        [ptr32, jnp.full((_PTR_BUF - (nseg + 1),), n, jnp.int32)])

    mesh = plsc.VectorSubcoreMesh(core_axis_name="c", subcore_axis_name="s",
                                  num_cores=_NC, num_subcores=_NS)
    sc_out = pl.kernel(
        _sc_segsum_body,
        out_type=jax.ShapeDtypeStruct((_NSEG_PAD + _NW, embed_dim),
                                      jnp.float32),
        mesh=mesh,
        scratch_types=[
            pltpu.VMEM((_PTR_BUF,), jnp.int32),
            pltpu.VMEM((_CH, embed_dim), jnp.float32),
            pltpu.VMEM((_CH, embed_dim), jnp.float32),
            pltpu.VMEM((_FB, embed_dim), jnp.float32),
            pltpu.VMEM((_FB,), jnp.int32),
            pltpu.SemaphoreType.DMA,
            pltpu.SemaphoreType.DMA,
        ],
    )(h, ptr_pad)
    return sc_out[:nseg]


# speedup vs baseline: 2.7066x; 1.1264x over previous
import functools

import jax
import jax.numpy as jnp
from jax import lax
from jax.experimental import pallas as pl
from jax.experimental.pallas import tpu as pltpu
from jax.experimental.pallas import tpu_sc as plsc

_LANES = 16
_CH = 128
_NC = 2
_NS = 16
_NW = _NC * _NS
_NSEG_PAD = 1024
_PTR_BUF = _NSEG_PAD + 2 * _LANES
_FB = 32


def _tc_mlp_body(w1x_ref, w1h_ref, w2_ref, b1_ref, b2_ref, x_ref, h_ref,
                 out_ref):
    xb = x_ref[...].astype(jnp.bfloat16)
    hb = h_ref[...].astype(jnp.bfloat16)
    hidden = jnp.maximum(
        jnp.dot(xb, w1x_ref[...], preferred_element_type=jnp.float32)
        + jnp.dot(hb, w1h_ref[...], preferred_element_type=jnp.float32)
        + b1_ref[...], 0.0)
    out_ref[...] = jnp.dot(hidden.astype(jnp.bfloat16), w2_ref[...],
                           preferred_element_type=jnp.float32) + b2_ref[...]


def _extract(vec_ref, j):
    v = vec_ref[pl.ds(j, _LANES)]
    return v[0]


def _count_below(ptr_v, thresh):
    base = jnp.int32(0)
    step = _NSEG_PAD
    while step >= 1:
        nxt = base + step
        safe = jnp.minimum(nxt, _PTR_BUF - _LANES)
        v = _extract(ptr_v, safe)
        take = jnp.logical_and(v < thresh, nxt <= _NSEG_PAD)
        base = jnp.where(take, nxt, base)
        step //= 2
    return base


def _sc_segsum_body(h_hbm, ptr_hbm, out_hbm, ptr_v, buf0_v, buf1_v, outbuf_v,
                    idx_v, sem0, sem1):
    wid = lax.axis_index("s") * _NC + lax.axis_index("c")
    n_rows = h_hbm.shape[0] - _CH
    rw = n_rows // _NW
    pltpu.sync_copy(ptr_hbm, ptr_v)

    t_lo = wid * rw
    t_hi = (wid + 1) * rw
    f_lo = _count_below(ptr_v, t_lo)
    f_hi = jnp.where(wid == _NW - 1, _NSEG_PAD, _count_below(ptr_v, t_hi))
    nsegs = f_hi - f_lo

    iota = lax.broadcasted_iota(jnp.int32, (_LANES,), 0)
    dump = _NSEG_PAD + wid

    def _flush(batch, count):
        base = f_lo + batch * _FB
        for half in range(_FB // _LANES):
            ids = base + half * _LANES + iota
            valid = (half * _LANES + iota) < count
            idx_v[pl.ds(half * _LANES, _LANES)] = jnp.where(valid, ids, dump)
        pltpu.sync_copy(outbuf_v, out_hbm.at[idx_v])

    def _fetch(off, buf, sem):
        pltpu.make_async_copy(h_hbm.at[pl.ds(off, _CH)], buf, sem).start()

    def _accum(buf, off, start, end, accs):
        rlo = jnp.maximum(start - off, 0)
        rhi = jnp.minimum(end - off, _CH)

        def row_body(r, accs):
            return tuple(accs[k] + buf[r, pl.ds(k * _LANES, _LANES)]
                         for k in range(8))

        return lax.fori_loop(rlo, rhi, row_body, accs)

    def seg_body(j, carry):
        seg = f_lo + j
        start = _extract(ptr_v, seg)
        end = _extract(ptr_v, seg + 1)
        abase = (start // 8) * 8
        nch = (end - abase + _CH - 1) // _CH

        @pl.when(nch > 0)
        def _():
            _fetch(abase, buf0_v, sem0)

        def pair_body(cc, accs):
            c0 = 2 * cc
            off0 = abase + c0 * _CH
            pltpu.make_async_copy(h_hbm.at[pl.ds(off0, _CH)], buf0_v,
                                  sem0).wait()
            odd = c0 + 1 < nch

            @pl.when(odd)
            def _():
                _fetch(off0 + _CH, buf1_v, sem1)

            accs = _accum(buf0_v, off0, start, end, accs)

            @pl.when(odd)
            def _():
                pltpu.make_async_copy(h_hbm.at[pl.ds(off0 + _CH, _CH)],
                                      buf1_v, sem1).wait()

            @pl.when(c0 + 2 < nch)
            def _():
                _fetch(off0 + 2 * _CH, buf0_v, sem0)

            return _accum(buf1_v, off0 + _CH, start, end, accs)

        zeros = tuple(jnp.zeros((_LANES,), jnp.float32) for _ in range(8))
        accs = lax.fori_loop(0, (nch + 1) // 2, pair_body, zeros)
        slot = jnp.bitwise_and(j, _FB - 1)
        for k in range(8):
            outbuf_v[slot, pl.ds(k * _LANES, _LANES)] = accs[k]

        @pl.when(slot == _FB - 1)
        def _():
            _flush(j // _FB, _FB)

        return carry

    lax.fori_loop(0, nsegs, seg_body, 0)
    rem = jnp.bitwise_and(nsegs, _FB - 1)

    @pl.when(rem > 0)
    def _():
        _flush(nsegs // _FB, rem)


def kernel(h_node, x, ptr, W1, b1, W2, b2):
    n, embed_dim = h_node.shape
    nfeat = x.shape[1]
    nseg = ptr.shape[0] - 1
    hidden_dim = W1.shape[1]

    block_rows = 16000
    grid = (n // block_rows,)
    w1x = W1[:nfeat].astype(jnp.bfloat16)
    w1h = W1[nfeat:].astype(jnp.bfloat16)
    w2 = W2.astype(jnp.bfloat16)

    h = pl.pallas_call(
        _tc_mlp_body,
        grid=grid,
        in_specs=[
            pl.BlockSpec((nfeat, hidden_dim), lambda i: (0, 0)),
            pl.BlockSpec((embed_dim, hidden_dim), lambda i: (0, 0)),
            pl.BlockSpec((hidden_dim, embed_dim), lambda i: (0, 0)),
            pl.BlockSpec((1, hidden_dim), lambda i: (0, 0)),
            pl.BlockSpec((1, embed_dim), lambda i: (0, 0)),
            pl.BlockSpec((block_rows, nfeat), lambda i: (i, 0)),
            pl.BlockSpec((block_rows, embed_dim), lambda i: (i, 0)),
        ],
        out_specs=pl.BlockSpec((block_rows, embed_dim), lambda i: (i, 0)),
        out_shape=jax.ShapeDtypeStruct((n + _CH, embed_dim), jnp.float32),
        compiler_params=pltpu.CompilerParams(
            dimension_semantics=("parallel",),
        ),
    )(w1x, w1h, w2, b1.reshape(1, -1), b2.reshape(1, -1), x, h_node)

    ptr32 = ptr.astype(jnp.int32)
    ptr_pad = jnp.concatenate(
        [ptr32, jnp.full((_PTR_BUF - (nseg + 1),), n, jnp.int32)])

    mesh = plsc.VectorSubcoreMesh(core_axis_name="c", subcore_axis_name="s",
                                  num_cores=_NC, num_subcores=_NS)
    sc_out = pl.kernel(
        _sc_segsum_body,
        out_type=jax.ShapeDtypeStruct((_NSEG_PAD + _NW, embed_dim),
                                      jnp.float32),
        mesh=mesh,
        scratch_types=[
            pltpu.VMEM((_PTR_BUF,), jnp.int32),
            pltpu.VMEM((_CH, embed_dim), jnp.float32),
            pltpu.VMEM((_CH, embed_dim), jnp.float32),
            pltpu.VMEM((_FB, embed_dim), jnp.float32),
            pltpu.VMEM((_FB,), jnp.int32),
            pltpu.SemaphoreType.DMA,
            pltpu.SemaphoreType.DMA,
        ],
    )(h, ptr_pad)
    return sc_out[:nseg]
